# baseline (speedup 1.0000x reference)
import functools

import jax
import jax.numpy as jnp
from jax import lax
from jax.experimental import pallas as pl
from jax.experimental.pallas import tpu as pltpu
from jax.experimental.pallas import tpu_sc as plsc

N = 10000
N_PAD = 10112
E = 320000
D_IN = 128
D_OUT = 128

NC = 2
NS = 16
NT = NC * NS
LANES = 16

CHUNK = 112
E_PAD = 329728
EDGES_PER_TILE = E_PAD // NT
NCHUNKS = EDGES_PER_TILE // CHUNK
NBUF = 3
ROWS_PER_TILE = N_PAD // NS

MM_BLK = 1264


def _matmul_body(x_ref, w_ref, o_ref):
    o_ref[...] = jnp.dot(x_ref[...], w_ref[...], preferred_element_type=jnp.float32)


def _tc_matmul(x, W):
    return pl.pallas_call(
        _matmul_body,
        grid=(N_PAD // MM_BLK,),
        in_specs=[
            pl.BlockSpec((MM_BLK, D_IN), lambda i: (i, 0)),
            pl.BlockSpec((D_IN, D_OUT), lambda i: (0, 0)),
        ],
        out_specs=pl.BlockSpec((MM_BLK, D_OUT), lambda i: (i, 0)),
        out_shape=jax.ShapeDtypeStruct((N_PAD, D_OUT), jnp.float32),
    )(x, W)


def _add_body(a_ref, b_ref, o_ref):
    o_ref[...] = a_ref[...] + b_ref[...]


def _tc_add(a, b):
    return pl.pallas_call(
        _add_body,
        grid=(N_PAD // MM_BLK,),
        in_specs=[
            pl.BlockSpec((MM_BLK, D_OUT), lambda i: (i, 0)),
            pl.BlockSpec((MM_BLK, D_OUT), lambda i: (i, 0)),
        ],
        out_specs=pl.BlockSpec((MM_BLK, D_OUT), lambda i: (i, 0)),
        out_shape=jax.ShapeDtypeStruct((N_PAD, D_OUT), jnp.float32),
    )(a, b)


def _bcast_lane(v, i):
    idx = jnp.full((LANES,), i, dtype=jnp.int32)
    return lax.gather(
        v,
        idx[:, None],
        dimension_numbers=lax.GatherDimensionNumbers(
            offset_dims=(), collapsed_slice_dims=(0,), start_index_map=(0,)
        ),
        slice_sizes=(1,),
        mode=lax.GatherScatterMode.PROMISE_IN_BOUNDS,
    )


def _sc_spmm_body(
    ps, rows_hbm, cols_hbm, vals_hbm, out0, out1,
    c0, c1, c2, r0, r1, r2, v0, v1, v2, b0, b1, b2, acc,
    cs0, cs1, cs2, rs0, rs1, rs2, is0, is1, is2,
    gs0, gs1, gs2, ss0, ss1, ss2, wsem,
):
    cc = lax.axis_index("c")
    s = lax.axis_index("s")
    tid = cc * NS + s

    cols = [c0, c1, c2]
    rows = [r0, r1, r2]
    vals = [v0, v1, v2]
    bufs = [b0, b1, b2]
    csem = [cs0, cs1, cs2]
    rsem = [rs0, rs1, rs2]
    isem = [is0, is1, is2]
    gsem = [gs0, gs1, gs2]
    ssem = [ss0, ss1, ss2]

    zero16 = jnp.zeros((LANES,), jnp.float32)

    def zrow(i, carry):
        for j in range(D_OUT // LANES):
            b0[i, pl.ds(j * LANES, LANES)] = zero16
        return carry

    lax.fori_loop(0, CHUNK, zrow, 0)
    row0 = s * ROWS_PER_TILE
    for b in range(5):
        pltpu.async_copy(b0, acc.at[pl.ds(row0 + b * CHUNK, CHUNK)], wsem)
    pltpu.async_copy(b0.at[pl.ds(0, 72)], acc.at[pl.ds(row0 + 5 * CHUNK, 72)], wsem)
    for b in range(5):
        pltpu.make_async_copy(b0, acc.at[pl.ds(row0, CHUNK)], wsem).wait()
    pltpu.make_async_copy(b0.at[pl.ds(0, 72)], acc.at[pl.ds(row0, 72)], wsem).wait()
    plsc.subcore_barrier()

    clamp = NCHUNKS - 1

    def cols_load(i, m):
        pltpu.async_copy(cols_hbm.at[tid, jnp.minimum(i, clamp)], cols[m], csem[m])

    def cols_wait(m):
        pltpu.make_async_copy(cols_hbm.at[0, 0], cols[m], csem[m]).wait()

    def rows_load(i, m):
        pltpu.async_copy(rows_hbm.at[tid, jnp.minimum(i, clamp)], rows[m], rsem[m])

    def rows_wait(m):
        pltpu.make_async_copy(rows_hbm.at[0, 0], rows[m], rsem[m]).wait()

    def vals_load(i, m):
        pltpu.async_copy(vals_hbm.at[tid, jnp.minimum(i, clamp)], vals[m], isem[m])

    def vals_wait(m):
        pltpu.make_async_copy(vals_hbm.at[0, 0], vals[m], isem[m]).wait()

    def gather(m):
        pltpu.async_copy(ps.at[cols[m]], bufs[m], gsem[m])

    def gather_wait(m):
        pltpu.make_async_copy(ps.at[pl.ds(0, CHUNK)], bufs[m], gsem[m]).wait()

    def scatter(m):
        pltpu.async_copy(bufs[m], acc.at[rows[m]], ssem[m], add=True)

    def scatter_wait(m):
        pltpu.make_async_copy(bufs[m], acc.at[pl.ds(0, CHUNK)], ssem[m]).wait()

    def scale(m):
        buf = bufs[m]
        val = vals[m]
        for g in range(CHUNK // LANES):
            vv = val[pl.ds(g * LANES, LANES)]
            for i in range(LANES):
                e = g * LANES + i
                vb = _bcast_lane(vv, i)
                for jf in range(D_OUT // LANES):
                    sl = pl.ds(jf * LANES, LANES)
                    buf[e, sl] = buf[e, sl] * vb

    for m in range(NBUF):
        cols_load(m, m)
        vals_load(m, m)
    rows_load(0, 0)
    rows_load(1, 1)
    cols_wait(0)
    gather(0)
    cols_wait(1)
    gather(1)

    cols_wait(2)
    gather(2)
    rows_load(2, 2)
    vals_wait(0)
    gather_wait(0)
    cols_load(3, 0)
    scale(0)
    vals_load(3, 0)
    rows_wait(0)
    scatter(0)

    scatter_wait(0)
    cols_wait(0)
    gather(0)
    rows_load(3, 0)
    vals_wait(1)
    gather_wait(1)
    cols_load(4, 1)
    scale(1)
    vals_load(4, 1)
    rows_wait(1)
    scatter(1)

    def sbody(j, cur, n2):
        scatter_wait(n2)
        cols_wait(n2)
        gather(n2)
        rows_load(j + 2, n2)
        vals_wait(cur)
        gather_wait(cur)
        cols_load(j + 4, cur)
        scale(cur)
        vals_load(j + 4, cur)
        rows_wait(cur)
        scatter(cur)

    def body(k, carry):
        j = 3 * k + 2
        sbody(j, 2, 1)
        sbody(j + 1, 0, 2)
        sbody(j + 2, 1, 0)
        return carry

    lax.fori_loop(0, (NCHUNKS - 5) // 3, body, 0)

    sbody(NCHUNKS - 3, 2, 1)

    for j, m in ((NCHUNKS - 2, 0), (NCHUNKS - 1, 1)):
        vals_wait(m)
        gather_wait(m)
        scale(m)
        rows_wait(m)
        scatter(m)

    for m in range(NBUF):
        scatter_wait(m)
    cols_wait(2)
    vals_wait(2)
    plsc.subcore_barrier()

    @pl.when(cc == 0)
    def _():
        pltpu.async_copy(acc.at[pl.ds(row0, 576)], out0.at[pl.ds(row0, 576)], wsem)
        pltpu.async_copy(acc.at[pl.ds(row0 + 576, 56)], out0.at[pl.ds(row0 + 576, 56)], wsem)
        pltpu.make_async_copy(acc.at[pl.ds(row0, 576)], out0.at[pl.ds(row0, 576)], wsem).wait()
        pltpu.make_async_copy(acc.at[pl.ds(row0, 56)], out0.at[pl.ds(row0, 56)], wsem).wait()

    @pl.when(cc == 1)
    def _():
        pltpu.async_copy(acc.at[pl.ds(row0, 576)], out1.at[pl.ds(row0, 576)], wsem)
        pltpu.async_copy(acc.at[pl.ds(row0 + 576, 56)], out1.at[pl.ds(row0 + 576, 56)], wsem)
        pltpu.make_async_copy(acc.at[pl.ds(row0, 576)], out1.at[pl.ds(row0, 576)], wsem).wait()
        pltpu.make_async_copy(acc.at[pl.ds(row0, 56)], out1.at[pl.ds(row0, 56)], wsem).wait()


_sc_spmm = functools.partial(
    pl.kernel,
    mesh=plsc.VectorSubcoreMesh(core_axis_name="c", subcore_axis_name="s"),
    out_type=[
        jax.ShapeDtypeStruct((N_PAD, D_OUT), jnp.float32),
        jax.ShapeDtypeStruct((N_PAD, D_OUT), jnp.float32),
    ],
    scratch_types=(
        [pltpu.VMEM((CHUNK,), jnp.int32) for _ in range(3)]
        + [pltpu.VMEM((CHUNK,), jnp.int32) for _ in range(3)]
        + [pltpu.VMEM((CHUNK,), jnp.float32) for _ in range(3)]
        + [pltpu.VMEM((CHUNK, D_OUT), jnp.float32) for _ in range(3)]
        + [pltpu.VMEM_SHARED((N_PAD, D_OUT), jnp.float32)]
        + [pltpu.SemaphoreType.DMA for _ in range(16)]
    ),
)(_sc_spmm_body)


def kernel(x, adj_indices, adj_values, W):
    x_pad = jnp.pad(x, ((0, N_PAD - N), (0, 0)))
    ps = _tc_matmul(x_pad, W)
    rows = jnp.pad(adj_indices[0], (0, E_PAD - E)).reshape(NT, NCHUNKS, CHUNK)
    cols = jnp.pad(adj_indices[1], (0, E_PAD - E)).reshape(NT, NCHUNKS, CHUNK)
    vals = jnp.pad(adj_values, (0, E_PAD - E)).reshape(NT, NCHUNKS, CHUNK)
    p0, p1 = _sc_spmm(ps, rows, cols, vals)
    return _tc_add(p0, p1)[:N]


# speedup vs baseline: 1.0016x; 1.0016x over previous
import functools

import jax
import jax.numpy as jnp
from jax import lax
from jax.experimental import pallas as pl
from jax.experimental.pallas import tpu as pltpu
from jax.experimental.pallas import tpu_sc as plsc

N = 10000
N_PAD = 10112
E = 320000
D_IN = 128
D_OUT = 128

NC = 2
NS = 16
NT = NC * NS
LANES = 16

CHUNK = 112
E_PAD = 329728
EDGES_PER_TILE = E_PAD // NT
NCHUNKS = EDGES_PER_TILE // CHUNK
NBUF = 3
ROWS_PER_TILE = N_PAD // NS

MM_BLK = 1264


def _matmul_body(x_ref, w_ref, o_ref):
    o_ref[...] = jnp.dot(x_ref[...], w_ref[...], preferred_element_type=jnp.float32)


def _tc_matmul(x, W):
    return pl.pallas_call(
        _matmul_body,
        grid=(N_PAD // MM_BLK,),
        in_specs=[
            pl.BlockSpec((MM_BLK, D_IN), lambda i: (i, 0)),
            pl.BlockSpec((D_IN, D_OUT), lambda i: (0, 0)),
        ],
        out_specs=pl.BlockSpec((MM_BLK, D_OUT), lambda i: (i, 0)),
        out_shape=jax.ShapeDtypeStruct((N_PAD, D_OUT), jnp.float32),
    )(x, W)


def _add_body(a_ref, b_ref, o_ref):
    o_ref[...] = a_ref[...] + b_ref[...]


def _tc_add(a, b):
    return pl.pallas_call(
        _add_body,
        grid=(N_PAD // MM_BLK,),
        in_specs=[
            pl.BlockSpec((MM_BLK, D_OUT), lambda i: (i, 0)),
            pl.BlockSpec((MM_BLK, D_OUT), lambda i: (i, 0)),
        ],
        out_specs=pl.BlockSpec((MM_BLK, D_OUT), lambda i: (i, 0)),
        out_shape=jax.ShapeDtypeStruct((N_PAD, D_OUT), jnp.float32),
    )(a, b)


def _bcast_lane(v, i):
    idx = jnp.full((LANES,), i, dtype=jnp.int32)
    return lax.gather(
        v,
        idx[:, None],
        dimension_numbers=lax.GatherDimensionNumbers(
            offset_dims=(), collapsed_slice_dims=(0,), start_index_map=(0,)
        ),
        slice_sizes=(1,),
        mode=lax.GatherScatterMode.PROMISE_IN_BOUNDS,
    )


def _sc_spmm_body(
    ps, rows_hbm, cols_hbm, vals_hbm, out0, out1,
    c0, c1, c2, r0, r1, r2, v0, v1, v2, b0, b1, b2, acc,
    cs0, cs1, cs2, rs0, rs1, rs2, is0, is1, is2,
    gs0, gs1, gs2, ss0, ss1, ss2, wsem,
):
    cc = lax.axis_index("c")
    s = lax.axis_index("s")
    tid = cc * NS + s

    cols = [c0, c1, c2]
    rows = [r0, r1, r2]
    vals = [v0, v1, v2]
    bufs = [b0, b1, b2]
    csem = [cs0, cs1, cs2]
    rsem = [rs0, rs1, rs2]
    isem = [is0, is1, is2]
    gsem = [gs0, gs1, gs2]
    ssem = [ss0, ss1, ss2]

    zero16 = jnp.zeros((LANES,), jnp.float32)

    def zrow(i, carry):
        for j in range(D_OUT // LANES):
            b0[i, pl.ds(j * LANES, LANES)] = zero16
        return carry

    lax.fori_loop(0, CHUNK, zrow, 0)
    row0 = s * ROWS_PER_TILE
    for b in range(5):
        pltpu.async_copy(b0, acc.at[pl.ds(row0 + b * CHUNK, CHUNK)], wsem)
    pltpu.async_copy(b0.at[pl.ds(0, 72)], acc.at[pl.ds(row0 + 5 * CHUNK, 72)], wsem)
    for b in range(5):
        pltpu.make_async_copy(b0, acc.at[pl.ds(row0, CHUNK)], wsem).wait()
    pltpu.make_async_copy(b0.at[pl.ds(0, 72)], acc.at[pl.ds(row0, 72)], wsem).wait()
    plsc.subcore_barrier()

    clamp = NCHUNKS - 1

    def cols_load(i, m):
        pltpu.async_copy(cols_hbm.at[tid, jnp.minimum(i, clamp)], cols[m], csem[m])

    def cols_wait(m):
        pltpu.make_async_copy(cols_hbm.at[0, 0], cols[m], csem[m]).wait()

    def rows_load(i, m):
        pltpu.async_copy(rows_hbm.at[tid, jnp.minimum(i, clamp)], rows[m], rsem[m])

    def rows_wait(m):
        pltpu.make_async_copy(rows_hbm.at[0, 0], rows[m], rsem[m]).wait()

    def vals_load(i, m):
        pltpu.async_copy(vals_hbm.at[tid, jnp.minimum(i, clamp)], vals[m], isem[m])

    def vals_wait(m):
        pltpu.make_async_copy(vals_hbm.at[0, 0], vals[m], isem[m]).wait()

    def gather(m):
        pltpu.async_copy(ps.at[cols[m]], bufs[m], gsem[m])

    def gather_wait(m):
        pltpu.make_async_copy(ps.at[pl.ds(0, CHUNK)], bufs[m], gsem[m]).wait()

    def scatter(m):
        pltpu.async_copy(bufs[m], acc.at[rows[m]], ssem[m], add=True)

    def scatter_wait(m):
        pltpu.make_async_copy(bufs[m], acc.at[pl.ds(0, CHUNK)], ssem[m]).wait()

    def scale(m):
        buf = bufs[m]
        val = vals[m]
        for g in range(CHUNK // LANES):
            vv = val[pl.ds(g * LANES, LANES)]
            for i in range(LANES):
                e = g * LANES + i
                vb = _bcast_lane(vv, i)
                for jf in range(D_OUT // LANES):
                    sl = pl.ds(jf * LANES, LANES)
                    buf[e, sl] = buf[e, sl] * vb

    for m in range(NBUF):
        cols_load(m, m)
        vals_load(m, m)
    rows_load(0, 0)
    rows_load(1, 1)
    cols_wait(0)
    gather(0)
    cols_wait(1)
    gather(1)

    cols_wait(2)
    gather(2)
    rows_load(2, 2)
    vals_wait(0)
    gather_wait(0)
    cols_load(3, 0)
    scale(0)
    vals_load(3, 0)
    rows_wait(0)
    scatter(0)

    scatter_wait(0)
    cols_wait(0)
    gather(0)
    rows_load(3, 0)
    vals_wait(1)
    gather_wait(1)
    cols_load(4, 1)
    scale(1)
    vals_load(4, 1)
    rows_wait(1)
    scatter(1)

    def sbody(j, cur, n2):
        scatter_wait(n2)
        cols_wait(n2)
        gather(n2)
        rows_load(j + 2, n2)
        vals_wait(cur)
        gather_wait(cur)
        cols_load(j + 3, cur)
        scale(cur)
        vals_load(j + 3, cur)
        rows_wait(cur)
        scatter(cur)

    def body(k, carry):
        j = 3 * k + 2
        sbody(j, 2, 1)
        sbody(j + 1, 0, 2)
        sbody(j + 2, 1, 0)
        return carry

    lax.fori_loop(0, (NCHUNKS - 5) // 3, body, 0)

    sbody(NCHUNKS - 3, 2, 1)

    for j, m in ((NCHUNKS - 2, 0), (NCHUNKS - 1, 1)):
        vals_wait(m)
        gather_wait(m)
        scale(m)
        rows_wait(m)
        scatter(m)

    for m in range(NBUF):
        scatter_wait(m)
    cols_wait(2)
    vals_wait(2)
    plsc.subcore_barrier()

    @pl.when(cc == 0)
    def _():
        pltpu.async_copy(acc.at[pl.ds(row0, 576)], out0.at[pl.ds(row0, 576)], wsem)
        pltpu.async_copy(acc.at[pl.ds(row0 + 576, 56)], out0.at[pl.ds(row0 + 576, 56)], wsem)
        pltpu.make_async_copy(acc.at[pl.ds(row0, 576)], out0.at[pl.ds(row0, 576)], wsem).wait()
        pltpu.make_async_copy(acc.at[pl.ds(row0, 56)], out0.at[pl.ds(row0, 56)], wsem).wait()

    @pl.when(cc == 1)
    def _():
        pltpu.async_copy(acc.at[pl.ds(row0, 576)], out1.at[pl.ds(row0, 576)], wsem)
        pltpu.async_copy(acc.at[pl.ds(row0 + 576, 56)], out1.at[pl.ds(row0 + 576, 56)], wsem)
        pltpu.make_async_copy(acc.at[pl.ds(row0, 576)], out1.at[pl.ds(row0, 576)], wsem).wait()
        pltpu.make_async_copy(acc.at[pl.ds(row0, 56)], out1.at[pl.ds(row0, 56)], wsem).wait()


_sc_spmm = functools.partial(
    pl.kernel,
    mesh=plsc.VectorSubcoreMesh(core_axis_name="c", subcore_axis_name="s"),
    out_type=[
        jax.ShapeDtypeStruct((N_PAD, D_OUT), jnp.float32),
        jax.ShapeDtypeStruct((N_PAD, D_OUT), jnp.float32),
    ],
    scratch_types=(
        [pltpu.VMEM((CHUNK,), jnp.int32) for _ in range(3)]
        + [pltpu.VMEM((CHUNK,), jnp.int32) for _ in range(3)]
        + [pltpu.VMEM((CHUNK,), jnp.float32) for _ in range(3)]
        + [pltpu.VMEM((CHUNK, D_OUT), jnp.float32) for _ in range(3)]
        + [pltpu.VMEM_SHARED((N_PAD, D_OUT), jnp.float32)]
        + [pltpu.SemaphoreType.DMA for _ in range(16)]
    ),
)(_sc_spmm_body)


def kernel(x, adj_indices, adj_values, W):
    x_pad = jnp.pad(x, ((0, N_PAD - N), (0, 0)))
    ps = _tc_matmul(x_pad, W)
    rows = jnp.pad(adj_indices[0], (0, E_PAD - E)).reshape(NT, NCHUNKS, CHUNK)
    cols = jnp.pad(adj_indices[1], (0, E_PAD - E)).reshape(NT, NCHUNKS, CHUNK)
    vals = jnp.pad(adj_values, (0, E_PAD - E)).reshape(NT, NCHUNKS, CHUNK)
    p0, p1 = _sc_spmm(ps, rows, cols, vals)
    return _tc_add(p0, p1)[:N]


# speedup vs baseline: 1.6246x; 1.6219x over previous
import functools

import jax
import jax.numpy as jnp
from jax import lax
from jax.experimental import pallas as pl
from jax.experimental.pallas import tpu as pltpu
from jax.experimental.pallas import tpu_sc as plsc

N = 10000
N_PAD = 10240
E = 320000
D_IN = 128
D_OUT = 128

NC = 2
NS = 16
LANES = 16

EDGES_PER_TILE = E // (NC * NS)
CHUNK = 80
NCHUNKS = EDGES_PER_TILE // CHUNK
ROWS_PER_TILE = N_PAD // NS
ZBLK = 128

MM_BLK = 1024


def _matmul_body(x_ref, w_ref, o_ref):
    o_ref[...] = jnp.dot(x_ref[...], w_ref[...], preferred_element_type=jnp.float32)


def _tc_matmul(x, W):
    return pl.pallas_call(
        _matmul_body,
        grid=(N_PAD // MM_BLK,),
        in_specs=[
            pl.BlockSpec((MM_BLK, D_IN), lambda i: (i, 0)),
            pl.BlockSpec((D_IN, D_OUT), lambda i: (0, 0)),
        ],
        out_specs=pl.BlockSpec((MM_BLK, D_OUT), lambda i: (i, 0)),
        out_shape=jax.ShapeDtypeStruct((N_PAD, D_OUT), jnp.float32),
    )(x, W)


def _add_body(a_ref, b_ref, o_ref):
    o_ref[...] = a_ref[...] + b_ref[...]


def _tc_add(a, b):
    return pl.pallas_call(
        _add_body,
        grid=(N_PAD // MM_BLK,),
        in_specs=[
            pl.BlockSpec((MM_BLK, D_OUT), lambda i: (i, 0)),
            pl.BlockSpec((MM_BLK, D_OUT), lambda i: (i, 0)),
        ],
        out_specs=pl.BlockSpec((MM_BLK, D_OUT), lambda i: (i, 0)),
        out_shape=jax.ShapeDtypeStruct((N_PAD, D_OUT), jnp.float32),
    )(a, b)


def _bcast_lane(v, i):
    idx = jnp.full((LANES,), i, dtype=jnp.int32)
    return lax.gather(
        v,
        idx[:, None],
        dimension_numbers=lax.GatherDimensionNumbers(
            offset_dims=(), collapsed_slice_dims=(0,), start_index_map=(0,)
        ),
        slice_sizes=(1,),
        mode=lax.GatherScatterMode.PROMISE_IN_BOUNDS,
    )


def _sc_spmm_body(
    ps, rows_hbm, cols_hbm, vals_hbm, out0, out1,
    cols_v, rows_v, vals_v, buf, zbuf, acc, sem
):
    c = lax.axis_index("c")
    s = lax.axis_index("s")

    zero16 = jnp.zeros((LANES,), jnp.float32)

    def zrow(i, carry):
        for j in range(D_OUT // LANES):
            zbuf[i, pl.ds(j * LANES, LANES)] = zero16
        return carry

    lax.fori_loop(0, ZBLK, zrow, 0)
    row0 = s * ROWS_PER_TILE
    for b in range(ROWS_PER_TILE // ZBLK):
        pltpu.sync_copy(zbuf, acc.at[pl.ds(row0 + b * ZBLK, ZBLK)])
    plsc.subcore_barrier()

    ebase = (c * NS + s) * EDGES_PER_TILE

    def body(it, carry):
        base = ebase + it * CHUNK
        pltpu.sync_copy(cols_hbm.at[pl.ds(base, CHUNK)], cols_v)
        pltpu.sync_copy(rows_hbm.at[pl.ds(base, CHUNK)], rows_v)
        pltpu.sync_copy(vals_hbm.at[pl.ds(base, CHUNK)], vals_v)

        pltpu.async_copy(ps.at[cols_v], buf, sem).wait()

        for g in range(CHUNK // LANES):
            vv = vals_v[pl.ds(g * LANES, LANES)]
            for i in range(LANES):
                e = g * LANES + i
                vb = _bcast_lane(vv, i)
                for j in range(D_OUT // LANES):
                    sl = pl.ds(j * LANES, LANES)
                    buf[e, sl] = buf[e, sl] * vb

        pltpu.sync_copy(buf, acc.at[rows_v], add=True)
        return carry

    lax.fori_loop(0, NCHUNKS, body, 0)
    plsc.subcore_barrier()

    @pl.when(c == 0)
    def _():
        for b in range(ROWS_PER_TILE // ZBLK):
            r = row0 + b * ZBLK
            pltpu.sync_copy(acc.at[pl.ds(r, ZBLK)], out0.at[pl.ds(r, ZBLK)])

    @pl.when(c == 1)
    def _():
        for b in range(ROWS_PER_TILE // ZBLK):
            r = row0 + b * ZBLK
            pltpu.sync_copy(acc.at[pl.ds(r, ZBLK)], out1.at[pl.ds(r, ZBLK)])


_sc_spmm = functools.partial(
    pl.kernel,
    mesh=plsc.VectorSubcoreMesh(core_axis_name="c", subcore_axis_name="s"),
    out_type=[
        jax.ShapeDtypeStruct((N_PAD, D_OUT), jnp.float32),
        jax.ShapeDtypeStruct((N_PAD, D_OUT), jnp.float32),
    ],
    scratch_types=[
        pltpu.VMEM((CHUNK,), jnp.int32),
        pltpu.VMEM((CHUNK,), jnp.int32),
        pltpu.VMEM((CHUNK,), jnp.float32),
        pltpu.VMEM((CHUNK, D_OUT), jnp.float32),
        pltpu.VMEM((ZBLK, D_OUT), jnp.float32),
        pltpu.VMEM_SHARED((N_PAD, D_OUT), jnp.float32),
        pltpu.SemaphoreType.DMA,
    ],
)(_sc_spmm_body)


def kernel(x, adj_indices, adj_values, W):
    x_pad = jnp.pad(x, ((0, N_PAD - N), (0, 0)))
    ps = _tc_matmul(x_pad, W)
    rows = adj_indices[0]
    cols = adj_indices[1]
    p0, p1 = _sc_spmm(ps, rows, cols, adj_values)
    return _tc_add(p0, p1)[:N]
